# baseline (speedup 1.0000x reference)
import functools

import jax
import jax.numpy as jnp
from jax import lax
from jax.experimental import pallas as pl
from jax.experimental.pallas import tpu as pltpu
from jax.experimental.pallas import tpu_sc as plsc

D_IN, D_SAE, T, B = 768, 4096, 8, 8
NF = T * D_SAE
KMAX = 256
SB = 1024


def _enc_body(x_ref, w_ref, b_ref, o_ref):
    for t in range(T):
        o_ref[:, t, :] = (
            jnp.dot(x_ref[:, t, :], w_ref[t], preferred_element_type=jnp.float32)
            + b_ref[0][None, :]
        )


def _encode(x, W_enc, b_enc2):
    return pl.pallas_call(
        _enc_body,
        grid=(D_SAE // SB,),
        in_specs=[
            pl.BlockSpec((B, T, D_IN), lambda s: (0, 0, 0)),
            pl.BlockSpec((T, D_IN, SB), lambda s: (0, 0, s)),
            pl.BlockSpec((1, SB), lambda s: (0, s)),
        ],
        out_specs=pl.BlockSpec((B, T, SB), lambda s: (0, 0, s)),
        out_shape=jax.ShapeDtypeStruct((B, T, D_SAE), jnp.float32),
    )(x, W_enc, b_enc2)


def _topk_body(k_ref, pre_ref, z_ref):
    pre = pre_ref[...]
    u = lax.bitcast_convert_type(pre, jnp.uint32)
    neg = (u >> 31) != 0
    key = jnp.where(neg, ~u, u | jnp.uint32(0x80000000))
    kk = jnp.minimum(k_ref[0], KMAX)

    def step(i, cur):
        bit = (jnp.uint32(1) << (jnp.uint32(31) - i.astype(jnp.uint32)))
        cand = cur | bit
        cnt = jnp.sum((key >= cand).astype(jnp.int32), axis=1, keepdims=True)
        return jnp.where(cnt >= kk, cand, cur)

    cur = lax.fori_loop(0, 32, step, jnp.zeros((B, 1), jnp.uint32))
    mask = key >= cur
    z_ref[...] = jnp.where(mask, jnp.maximum(pre, 0.0), 0.0)


def _topk_z(pre_flat, k_arr):
    return pl.pallas_call(
        _topk_body,
        in_specs=[
            pl.BlockSpec(memory_space=pltpu.SMEM),
            pl.BlockSpec(memory_space=pltpu.VMEM),
        ],
        out_specs=pl.BlockSpec(memory_space=pltpu.VMEM),
        out_shape=jax.ShapeDtypeStruct((B, NF), jnp.float32),
    )(k_arr, pre_flat)


NC, NS, L = 2, 16, 16
G = 16
NV = D_SAE // L
NGMAX = KMAX // G


CHUNK = 2 * D_SAE
NVC = CHUNK // L


def _sc_dec_body(z_hbm, w_hbm, o_hbm,
                 zbuf, idx_buf, val_buf, rows, acc, sem):
    wid = lax.axis_index("s") * NC + lax.axis_index("c")
    b = wid >> 2
    tpair = (wid & 3) * 2
    iota = lax.iota(jnp.int32, L)

    pltpu.sync_copy(z_hbm.at[pl.ds(b * NF + tpair * D_SAE, CHUNK)], zbuf)

    pad_i = jnp.full((L,), tpair, jnp.int32)
    zero_f = jnp.zeros((L,), jnp.float32)
    for j in range(NGMAX + 1):
        idx_buf[j, :] = pad_i
        val_buf[pl.ds(j * L, L)] = zero_f

    U = 4

    def cbody(q, off):
        vs = [zbuf[pl.ds((q * U + i) * L, L)] for i in range(U)]
        ms = [v > 0.0 for v in vs]
        mo = ms[0]
        for i in range(1, U):
            mo = mo | ms[i]
        anyb = plsc.all_reduce_population_count(mo)[0] > 0

        def nonempty(o):
            for i in range(U):
                l_ids = (q * U + i) * L + iota
                r_ids = (l_ids & (D_SAE - 1)) * T + tpair + (l_ids >> 12)
                pos = o + plsc.cumsum(ms[i].astype(jnp.int32)) - 1
                plsc.store_scatter(idx_buf, [pos // G, pos % G], r_ids, mask=ms[i])
                plsc.store_scatter(val_buf, [pos], vs[i], mask=ms[i])
                o = o + plsc.all_reduce_population_count(ms[i])[0]
            return o

        return lax.cond(anyb, nonempty, lambda o: o, off)

    off0 = lax.fori_loop(0, NVC // 2 // U, cbody, jnp.int32(0))
    padp = off0 + iota
    plsc.store_scatter(idx_buf, [padp // G, padp % G], pad_i)
    plsc.store_scatter(val_buf, [padp], zero_f)
    off0r = (off0 + (G - 1)) & ~(G - 1)
    ng0 = off0r >> 4
    off = lax.fori_loop(NVC // 2 // U, NVC // U, cbody, off0r)

    for c in range(2 * D_IN // L):
        acc[pl.ds(c * L, L)] = zero_f

    ngrp = (off + (G - 1)) // G
    NB = 8

    def round_body(r, carry):
        gbase = r * NB
        gn = jnp.minimum(ngrp - gbase, NB)

        def fire(g2, c2):
            pltpu.async_copy(w_hbm.at[idx_buf.at[gbase + g2]], rows.at[g2], sem)
            return c2

        lax.fori_loop(0, gn, fire, jnp.int32(0))

        dn = lax.GatherDimensionNumbers(
            offset_dims=(), collapsed_slice_dims=(0,), start_index_map=(0,)
        )

        def drain_acc(g2, c2):
            pltpu.make_async_copy(
                w_hbm.at[idx_buf.at[0]], rows.at[g2], sem
            ).wait()
            gg = gbase + g2
            tbase = jnp.where(gg >= ng0, D_IN, 0)
            vgrp = val_buf[pl.ds(gg * G, G)]
            vjs = [
                lax.gather(
                    vgrp, jnp.full((L, 1), j, jnp.int32), dn, (1,),
                    mode=lax.GatherScatterMode.PROMISE_IN_BOUNDS,
                )
                for j in range(G)
            ]
            for c in range(D_IN // L):
                a = acc[pl.ds(tbase + c * L, L)]
                for j in range(G):
                    a = a + vjs[j] * rows[g2, j, pl.ds(c * L, L)]
                acc[pl.ds(tbase + c * L, L)] = a
            return c2

        lax.fori_loop(0, gn, drain_acc, jnp.int32(0))
        return carry

    lax.fori_loop(0, (ngrp + (NB - 1)) // NB, round_body, jnp.int32(0))

    pltpu.sync_copy(acc, o_hbm.at[pl.ds((b * T + tpair) * D_IN, 2 * D_IN)])


def _decode_sc(z_flat, W_dec):
    z1d = z_flat.reshape(B * NF)
    w2d = W_dec.reshape(D_SAE * T, D_IN)
    mesh = plsc.VectorSubcoreMesh(core_axis_name="c", subcore_axis_name="s")
    fn = pl.kernel(
        _sc_dec_body,
        out_type=jax.ShapeDtypeStruct((B * T * D_IN,), jnp.float32),
        mesh=mesh,
        compiler_params=pltpu.CompilerParams(needs_layout_passes=False),
        scratch_types=[
            pltpu.VMEM((CHUNK,), jnp.float32),
            pltpu.VMEM((NGMAX + 1, G), jnp.int32),
            pltpu.VMEM((KMAX + G,), jnp.float32),
            pltpu.VMEM((8, G, D_IN), jnp.float32),
            pltpu.VMEM((2 * D_IN,), jnp.float32),
            pltpu.SemaphoreType.DMA,
        ],
    )
    return fn(z1d, w2d).reshape(B, T, D_IN)


def _fin_body(p_ref, b_ref, x_ref, xh_ref, loss_ref):
    xh = p_ref[...] + b_ref[...][None]
    xh_ref[...] = xh
    d = xh - x_ref[...]
    loss_ref[0, 0] = jnp.sum(d * d) / (B * T)


def _finalize(partial, b_dec, x):
    return pl.pallas_call(
        _fin_body,
        out_specs=(
            pl.BlockSpec(memory_space=pltpu.VMEM),
            pl.BlockSpec(memory_space=pltpu.SMEM),
        ),
        out_shape=(
            jax.ShapeDtypeStruct((B, T, D_IN), jnp.float32),
            jax.ShapeDtypeStruct((1, 1), jnp.float32),
        ),
    )(partial, b_dec, x)


def kernel(x, W_enc, W_dec, b_enc, b_dec, k):
    b_enc2 = b_enc.reshape(1, D_SAE)
    k_arr = jnp.asarray(k, jnp.int32).reshape(1)
    pre = _encode(x, W_enc, b_enc2)
    z_flat = _topk_z(pre.reshape(B, NF), k_arr)
    z = z_flat.reshape(B, T, D_SAE)
    partial = _decode_sc(z_flat, W_dec)
    x_hat, loss = _finalize(partial, b_dec, x)
    return (loss.reshape(()), x_hat, z)


# speedup vs baseline: 1.0356x; 1.0356x over previous
import functools

import jax
import jax.numpy as jnp
from jax import lax
from jax.experimental import pallas as pl
from jax.experimental.pallas import tpu as pltpu
from jax.experimental.pallas import tpu_sc as plsc

D_IN, D_SAE, T, B = 768, 4096, 8, 8
NF = T * D_SAE
KMAX = 256
SB = 256


def _enc_body(x_ref, w_ref, b_ref, o_ref):
    for t in range(T):
        o_ref[:, t, :] = (
            jnp.dot(x_ref[:, t, :], w_ref[t], preferred_element_type=jnp.float32)
            + b_ref[0][None, :]
        )


def _encode(x, W_enc, b_enc2):
    return pl.pallas_call(
        _enc_body,
        grid=(D_SAE // SB,),
        in_specs=[
            pl.BlockSpec((B, T, D_IN), lambda s: (0, 0, 0)),
            pl.BlockSpec((T, D_IN, SB), lambda s: (0, 0, s)),
            pl.BlockSpec((1, SB), lambda s: (0, s)),
        ],
        out_specs=pl.BlockSpec((B, T, SB), lambda s: (0, 0, s)),
        out_shape=jax.ShapeDtypeStruct((B, T, D_SAE), jnp.float32),
    )(x, W_enc, b_enc2)


def _topk_body(k_ref, pre_ref, z_ref):
    pre = pre_ref[...]
    u = lax.bitcast_convert_type(pre, jnp.uint32)
    neg = (u >> 31) != 0
    key = jnp.where(neg, ~u, u | jnp.uint32(0x80000000))
    kk = jnp.minimum(k_ref[0], KMAX)

    def step(i, cur):
        bit = (jnp.uint32(1) << (jnp.uint32(31) - i.astype(jnp.uint32)))
        cand = cur | bit
        cnt = jnp.sum((key >= cand).astype(jnp.int32), axis=1, keepdims=True)
        return jnp.where(cnt >= kk, cand, cur)

    cur = lax.fori_loop(0, 32, step, jnp.zeros((B, 1), jnp.uint32))
    mask = key >= cur
    z_ref[...] = jnp.where(mask, jnp.maximum(pre, 0.0), 0.0)


def _topk_z(pre_flat, k_arr):
    return pl.pallas_call(
        _topk_body,
        in_specs=[
            pl.BlockSpec(memory_space=pltpu.SMEM),
            pl.BlockSpec(memory_space=pltpu.VMEM),
        ],
        out_specs=pl.BlockSpec(memory_space=pltpu.VMEM),
        out_shape=jax.ShapeDtypeStruct((B, NF), jnp.float32),
    )(k_arr, pre_flat)


NC, NS, L = 2, 16, 16
G = 16
NV = D_SAE // L
NGMAX = KMAX // G


CHUNK = 2 * D_SAE
NVC = CHUNK // L


def _sc_dec_body(z_hbm, w_hbm, o_hbm,
                 zbuf, idx_buf, val_buf, rows, acc, sem):
    wid = lax.axis_index("s") * NC + lax.axis_index("c")
    b = wid >> 2
    tpair = (wid & 3) * 2
    iota = lax.iota(jnp.int32, L)

    pltpu.sync_copy(z_hbm.at[pl.ds(b * NF + tpair * D_SAE, CHUNK)], zbuf)

    pad_i = jnp.full((L,), tpair, jnp.int32)
    zero_f = jnp.zeros((L,), jnp.float32)
    for j in range(NGMAX + 1):
        idx_buf[j, :] = pad_i
        val_buf[pl.ds(j * L, L)] = zero_f

    U = 4

    def cbody(q, off):
        vs = [zbuf[pl.ds((q * U + i) * L, L)] for i in range(U)]
        ms = [v > 0.0 for v in vs]
        mo = ms[0]
        for i in range(1, U):
            mo = mo | ms[i]
        anyb = plsc.all_reduce_population_count(mo)[0] > 0

        def nonempty(o):
            for i in range(U):
                l_ids = (q * U + i) * L + iota
                r_ids = (l_ids & (D_SAE - 1)) * T + tpair + (l_ids >> 12)
                pos = o + plsc.cumsum(ms[i].astype(jnp.int32)) - 1
                plsc.store_scatter(idx_buf, [pos // G, pos % G], r_ids, mask=ms[i])
                plsc.store_scatter(val_buf, [pos], vs[i], mask=ms[i])
                o = o + plsc.all_reduce_population_count(ms[i])[0]
            return o

        return lax.cond(anyb, nonempty, lambda o: o, off)

    off0 = lax.fori_loop(0, NVC // 2 // U, cbody, jnp.int32(0))
    padp = off0 + iota
    plsc.store_scatter(idx_buf, [padp // G, padp % G], pad_i)
    plsc.store_scatter(val_buf, [padp], zero_f)
    off0r = (off0 + (G - 1)) & ~(G - 1)
    ng0 = off0r >> 4
    off = lax.fori_loop(NVC // 2 // U, NVC // U, cbody, off0r)

    for c in range(2 * D_IN // L):
        acc[pl.ds(c * L, L)] = zero_f

    ngrp = (off + (G - 1)) // G
    NB = 8

    def round_body(r, carry):
        gbase = r * NB
        gn = jnp.minimum(ngrp - gbase, NB)

        def fire(g2, c2):
            pltpu.async_copy(w_hbm.at[idx_buf.at[gbase + g2]], rows.at[g2], sem)
            return c2

        lax.fori_loop(0, gn, fire, jnp.int32(0))

        dn = lax.GatherDimensionNumbers(
            offset_dims=(), collapsed_slice_dims=(0,), start_index_map=(0,)
        )

        def drain_acc(g2, c2):
            pltpu.make_async_copy(
                w_hbm.at[idx_buf.at[0]], rows.at[g2], sem
            ).wait()
            gg = gbase + g2
            tbase = jnp.where(gg >= ng0, D_IN, 0)
            vgrp = val_buf[pl.ds(gg * G, G)]
            vjs = [
                lax.gather(
                    vgrp, jnp.full((L, 1), j, jnp.int32), dn, (1,),
                    mode=lax.GatherScatterMode.PROMISE_IN_BOUNDS,
                )
                for j in range(G)
            ]
            for c in range(D_IN // L):
                a = acc[pl.ds(tbase + c * L, L)]
                for j in range(G):
                    a = a + vjs[j] * rows[g2, j, pl.ds(c * L, L)]
                acc[pl.ds(tbase + c * L, L)] = a
            return c2

        lax.fori_loop(0, gn, drain_acc, jnp.int32(0))
        return carry

    lax.fori_loop(0, (ngrp + (NB - 1)) // NB, round_body, jnp.int32(0))

    pltpu.sync_copy(acc, o_hbm.at[pl.ds((b * T + tpair) * D_IN, 2 * D_IN)])


def _decode_sc(z_flat, W_dec):
    z1d = z_flat.reshape(B * NF)
    w2d = W_dec.reshape(D_SAE * T, D_IN)
    mesh = plsc.VectorSubcoreMesh(core_axis_name="c", subcore_axis_name="s")
    fn = pl.kernel(
        _sc_dec_body,
        out_type=jax.ShapeDtypeStruct((B * T * D_IN,), jnp.float32),
        mesh=mesh,
        compiler_params=pltpu.CompilerParams(needs_layout_passes=False),
        scratch_types=[
            pltpu.VMEM((CHUNK,), jnp.float32),
            pltpu.VMEM((NGMAX + 1, G), jnp.int32),
            pltpu.VMEM((KMAX + G,), jnp.float32),
            pltpu.VMEM((8, G, D_IN), jnp.float32),
            pltpu.VMEM((2 * D_IN,), jnp.float32),
            pltpu.SemaphoreType.DMA,
        ],
    )
    return fn(z1d, w2d).reshape(B, T, D_IN)


def _fin_body(p_ref, b_ref, x_ref, xh_ref, loss_ref):
    xh = p_ref[...] + b_ref[...][None]
    xh_ref[...] = xh
    d = xh - x_ref[...]
    loss_ref[0, 0] = jnp.sum(d * d) / (B * T)


def _finalize(partial, b_dec, x):
    return pl.pallas_call(
        _fin_body,
        out_specs=(
            pl.BlockSpec(memory_space=pltpu.VMEM),
            pl.BlockSpec(memory_space=pltpu.SMEM),
        ),
        out_shape=(
            jax.ShapeDtypeStruct((B, T, D_IN), jnp.float32),
            jax.ShapeDtypeStruct((1, 1), jnp.float32),
        ),
    )(partial, b_dec, x)


def kernel(x, W_enc, W_dec, b_enc, b_dec, k):
    b_enc2 = b_enc.reshape(1, D_SAE)
    k_arr = jnp.asarray(k, jnp.int32).reshape(1)
    pre = _encode(x, W_enc, b_enc2)
    z_flat = _topk_z(pre.reshape(B, NF), k_arr)
    z = z_flat.reshape(B, T, D_SAE)
    partial = _decode_sc(z_flat, W_dec)
    x_hat, loss = _finalize(partial, b_dec, x)
    return (loss.reshape(()), x_hat, z)


# speedup vs baseline: 1.1403x; 1.1012x over previous
import functools

import jax
import jax.numpy as jnp
from jax import lax
from jax.experimental import pallas as pl
from jax.experimental.pallas import tpu as pltpu
from jax.experimental.pallas import tpu_sc as plsc

D_IN, D_SAE, T, B = 768, 4096, 8, 8
NF = T * D_SAE
KMAX = 256
SB = 256


def _enc_topk_body(k_ref, x_ref, w_ref, b_ref, z_ref, pre_ref):
    s = pl.program_id(0)
    for t in range(T):
        pre_ref[:, t, pl.ds(s * SB, SB)] = (
            jnp.dot(x_ref[:, t, :], w_ref[t], preferred_element_type=jnp.float32)
            + b_ref[0][None, :]
        )

    @pl.when(s == D_SAE // SB - 1)
    def _():
        pre = pre_ref[...]
        u = lax.bitcast_convert_type(pre, jnp.uint32)
        neg = (u >> 31) != 0
        key = jnp.where(neg, ~u, u | jnp.uint32(0x80000000))
        kk = jnp.minimum(k_ref[0], KMAX)

        def step(i, cur):
            bit = jnp.uint32(1) << (jnp.uint32(31) - i.astype(jnp.uint32))
            cand = cur | bit
            ge = (key >= cand).astype(jnp.int32)
            cnt = jnp.sum(
                jnp.sum(ge, axis=2, keepdims=True), axis=1, keepdims=True
            )
            return jnp.where(cnt >= kk, cand, cur)

        cur = lax.fori_loop(0, 32, step, jnp.zeros((B, 1, 1), jnp.uint32))
        z_ref[...] = jnp.where(key >= cur, jnp.maximum(pre, 0.0), 0.0)


def _encode_topk(x, W_enc, b_enc2, k_arr):
    return pl.pallas_call(
        _enc_topk_body,
        grid=(D_SAE // SB,),
        in_specs=[
            pl.BlockSpec(memory_space=pltpu.SMEM),
            pl.BlockSpec((B, T, D_IN), lambda s: (0, 0, 0)),
            pl.BlockSpec((T, D_IN, SB), lambda s: (0, 0, s)),
            pl.BlockSpec((1, SB), lambda s: (0, s)),
        ],
        out_specs=pl.BlockSpec((B, T, D_SAE), lambda s: (0, 0, 0)),
        out_shape=jax.ShapeDtypeStruct((B, T, D_SAE), jnp.float32),
        scratch_shapes=[pltpu.VMEM((B, T, D_SAE), jnp.float32)],
    )(k_arr, x, W_enc, b_enc2)


def _topk_body(k_ref, pre_ref, z_ref):
    pre = pre_ref[...]
    u = lax.bitcast_convert_type(pre, jnp.uint32)
    neg = (u >> 31) != 0
    key = jnp.where(neg, ~u, u | jnp.uint32(0x80000000))
    kk = jnp.minimum(k_ref[0], KMAX)

    def step(i, cur):
        bit = (jnp.uint32(1) << (jnp.uint32(31) - i.astype(jnp.uint32)))
        cand = cur | bit
        cnt = jnp.sum((key >= cand).astype(jnp.int32), axis=1, keepdims=True)
        return jnp.where(cnt >= kk, cand, cur)

    cur = lax.fori_loop(0, 32, step, jnp.zeros((B, 1), jnp.uint32))
    mask = key >= cur
    z_ref[...] = jnp.where(mask, jnp.maximum(pre, 0.0), 0.0)


def _topk_z(pre_flat, k_arr):
    return pl.pallas_call(
        _topk_body,
        in_specs=[
            pl.BlockSpec(memory_space=pltpu.SMEM),
            pl.BlockSpec(memory_space=pltpu.VMEM),
        ],
        out_specs=pl.BlockSpec(memory_space=pltpu.VMEM),
        out_shape=jax.ShapeDtypeStruct((B, NF), jnp.float32),
    )(k_arr, pre_flat)


NC, NS, L = 2, 16, 16
G = 16
NV = D_SAE // L
NGMAX = KMAX // G


CHUNK = 2 * D_SAE
NVC = CHUNK // L


def _sc_dec_body(z_hbm, w_hbm, o_hbm,
                 zbuf, idx_buf, val_buf, rows, acc, sem):
    wid = lax.axis_index("s") * NC + lax.axis_index("c")
    b = wid >> 2
    tpair = (wid & 3) * 2
    iota = lax.iota(jnp.int32, L)

    pltpu.sync_copy(z_hbm.at[pl.ds(b * NF + tpair * D_SAE, CHUNK)], zbuf)

    pad_i = jnp.full((L,), tpair, jnp.int32)
    zero_f = jnp.zeros((L,), jnp.float32)
    for j in range(NGMAX + 1):
        idx_buf[j, :] = pad_i
        val_buf[pl.ds(j * L, L)] = zero_f

    U = 4

    def cbody(q, off):
        vs = [zbuf[pl.ds((q * U + i) * L, L)] for i in range(U)]
        ms = [v > 0.0 for v in vs]
        mo = ms[0]
        for i in range(1, U):
            mo = mo | ms[i]
        anyb = plsc.all_reduce_population_count(mo)[0] > 0

        def nonempty(o):
            for i in range(U):
                l_ids = (q * U + i) * L + iota
                r_ids = (l_ids & (D_SAE - 1)) * T + tpair + (l_ids >> 12)
                pos = o + plsc.cumsum(ms[i].astype(jnp.int32)) - 1
                plsc.store_scatter(idx_buf, [pos // G, pos % G], r_ids, mask=ms[i])
                plsc.store_scatter(val_buf, [pos], vs[i], mask=ms[i])
                o = o + plsc.all_reduce_population_count(ms[i])[0]
            return o

        return lax.cond(anyb, nonempty, lambda o: o, off)

    off0 = lax.fori_loop(0, NVC // 2 // U, cbody, jnp.int32(0))
    padp = off0 + iota
    plsc.store_scatter(idx_buf, [padp // G, padp % G], pad_i)
    plsc.store_scatter(val_buf, [padp], zero_f)
    off0r = (off0 + (G - 1)) & ~(G - 1)
    ng0 = off0r >> 4
    off = lax.fori_loop(NVC // 2 // U, NVC // U, cbody, off0r)

    for c in range(2 * D_IN // L):
        acc[pl.ds(c * L, L)] = zero_f

    ngrp = (off + (G - 1)) // G
    NB = 8

    def round_body(r, carry):
        gbase = r * NB
        gn = jnp.minimum(ngrp - gbase, NB)

        def fire(g2, c2):
            pltpu.async_copy(w_hbm.at[idx_buf.at[gbase + g2]], rows.at[g2], sem)
            return c2

        lax.fori_loop(0, gn, fire, jnp.int32(0))

        dn = lax.GatherDimensionNumbers(
            offset_dims=(), collapsed_slice_dims=(0,), start_index_map=(0,)
        )

        def drain_acc(g2, c2):
            pltpu.make_async_copy(
                w_hbm.at[idx_buf.at[0]], rows.at[g2], sem
            ).wait()
            gg = gbase + g2
            tbase = jnp.where(gg >= ng0, D_IN, 0)
            vgrp = val_buf[pl.ds(gg * G, G)]
            vjs = [
                lax.gather(
                    vgrp, jnp.full((L, 1), j, jnp.int32), dn, (1,),
                    mode=lax.GatherScatterMode.PROMISE_IN_BOUNDS,
                )
                for j in range(G)
            ]
            for c in range(D_IN // L):
                a = acc[pl.ds(tbase + c * L, L)]
                for j in range(G):
                    a = a + vjs[j] * rows[g2, j, pl.ds(c * L, L)]
                acc[pl.ds(tbase + c * L, L)] = a
            return c2

        lax.fori_loop(0, gn, drain_acc, jnp.int32(0))
        return carry

    lax.fori_loop(0, (ngrp + (NB - 1)) // NB, round_body, jnp.int32(0))

    pltpu.sync_copy(acc, o_hbm.at[pl.ds((b * T + tpair) * D_IN, 2 * D_IN)])


def _decode_sc(z_flat, W_dec):
    z1d = z_flat.reshape(B * NF)
    w2d = W_dec.reshape(D_SAE * T, D_IN)
    mesh = plsc.VectorSubcoreMesh(core_axis_name="c", subcore_axis_name="s")
    fn = pl.kernel(
        _sc_dec_body,
        out_type=jax.ShapeDtypeStruct((B * T * D_IN,), jnp.float32),
        mesh=mesh,
        compiler_params=pltpu.CompilerParams(needs_layout_passes=False),
        scratch_types=[
            pltpu.VMEM((CHUNK,), jnp.float32),
            pltpu.VMEM((NGMAX + 1, G), jnp.int32),
            pltpu.VMEM((KMAX + G,), jnp.float32),
            pltpu.VMEM((8, G, D_IN), jnp.float32),
            pltpu.VMEM((2 * D_IN,), jnp.float32),
            pltpu.SemaphoreType.DMA,
        ],
    )
    return fn(z1d, w2d).reshape(B, T, D_IN)


def _fin_body(p_ref, b_ref, x_ref, xh_ref, loss_ref):
    xh = p_ref[...] + b_ref[...][None]
    xh_ref[...] = xh
    d = xh - x_ref[...]
    loss_ref[0, 0] = jnp.sum(d * d) / (B * T)


def _finalize(partial, b_dec, x):
    return pl.pallas_call(
        _fin_body,
        out_specs=(
            pl.BlockSpec(memory_space=pltpu.VMEM),
            pl.BlockSpec(memory_space=pltpu.SMEM),
        ),
        out_shape=(
            jax.ShapeDtypeStruct((B, T, D_IN), jnp.float32),
            jax.ShapeDtypeStruct((1, 1), jnp.float32),
        ),
    )(partial, b_dec, x)


def kernel(x, W_enc, W_dec, b_enc, b_dec, k):
    b_enc2 = b_enc.reshape(1, D_SAE)
    k_arr = jnp.asarray(k, jnp.int32).reshape(1)
    z = _encode_topk(x, W_enc, b_enc2, k_arr)
    partial = _decode_sc(z.reshape(B, NF), W_dec)
    x_hat, loss = _finalize(partial, b_dec, x)
    return (loss.reshape(()), x_hat, z)


# speedup vs baseline: 1.1732x; 1.0288x over previous
import functools

import jax
import jax.numpy as jnp
from jax import lax
from jax.experimental import pallas as pl
from jax.experimental.pallas import tpu as pltpu
from jax.experimental.pallas import tpu_sc as plsc

D_IN, D_SAE, T, B = 768, 4096, 8, 8
NF = T * D_SAE
KMAX = 256
SB = 256


def _enc_topk_body(k_ref, x_ref, w_ref, b_ref, z_ref, pre_ref):
    s = pl.program_id(0)
    for t in range(T):
        pre_ref[:, t, pl.ds(s * SB, SB)] = (
            jnp.dot(x_ref[:, t, :], w_ref[t], preferred_element_type=jnp.float32)
            + b_ref[0][None, :]
        )

    @pl.when(s == D_SAE // SB - 1)
    def _():
        pre = pre_ref[...]
        u = lax.bitcast_convert_type(pre, jnp.uint32)
        neg = (u >> 31) != 0
        key = jnp.where(neg, ~u, u | jnp.uint32(0x80000000))
        kk = jnp.minimum(k_ref[0], KMAX)

        def step(i, cur):
            bit = jnp.uint32(1) << (jnp.uint32(31) - i.astype(jnp.uint32))
            cand = cur | bit
            ge = (key >= cand).astype(jnp.int32)
            cnt = jnp.sum(
                jnp.sum(ge, axis=2, keepdims=True), axis=1, keepdims=True
            )
            return jnp.where(cnt >= kk, cand, cur)

        cur = lax.fori_loop(0, 32, step, jnp.zeros((B, 1, 1), jnp.uint32))
        z_ref[...] = jnp.where(key >= cur, jnp.maximum(pre, 0.0), 0.0)


def _encode_topk(x, W_enc, b_enc2, k_arr):
    return pl.pallas_call(
        _enc_topk_body,
        grid=(D_SAE // SB,),
        in_specs=[
            pl.BlockSpec(memory_space=pltpu.SMEM),
            pl.BlockSpec((B, T, D_IN), lambda s: (0, 0, 0)),
            pl.BlockSpec((T, D_IN, SB), lambda s: (0, 0, s)),
            pl.BlockSpec((1, SB), lambda s: (0, s)),
        ],
        out_specs=pl.BlockSpec((B, T, D_SAE), lambda s: (0, 0, 0)),
        out_shape=jax.ShapeDtypeStruct((B, T, D_SAE), jnp.float32),
        scratch_shapes=[pltpu.VMEM((B, T, D_SAE), jnp.float32)],
    )(k_arr, x, W_enc, b_enc2)


def _topk_body(k_ref, pre_ref, z_ref):
    pre = pre_ref[...]
    u = lax.bitcast_convert_type(pre, jnp.uint32)
    neg = (u >> 31) != 0
    key = jnp.where(neg, ~u, u | jnp.uint32(0x80000000))
    kk = jnp.minimum(k_ref[0], KMAX)

    def step(i, cur):
        bit = (jnp.uint32(1) << (jnp.uint32(31) - i.astype(jnp.uint32)))
        cand = cur | bit
        cnt = jnp.sum((key >= cand).astype(jnp.int32), axis=1, keepdims=True)
        return jnp.where(cnt >= kk, cand, cur)

    cur = lax.fori_loop(0, 32, step, jnp.zeros((B, 1), jnp.uint32))
    mask = key >= cur
    z_ref[...] = jnp.where(mask, jnp.maximum(pre, 0.0), 0.0)


def _topk_z(pre_flat, k_arr):
    return pl.pallas_call(
        _topk_body,
        in_specs=[
            pl.BlockSpec(memory_space=pltpu.SMEM),
            pl.BlockSpec(memory_space=pltpu.VMEM),
        ],
        out_specs=pl.BlockSpec(memory_space=pltpu.VMEM),
        out_shape=jax.ShapeDtypeStruct((B, NF), jnp.float32),
    )(k_arr, pre_flat)


NC, NS, L = 2, 16, 16
G = 16
NV = D_SAE // L
NGMAX = KMAX // G


CHUNK = 2 * D_SAE
NVC = CHUNK // L


def _sc_dec_body(z_hbm, w_hbm, o_hbm,
                 zbuf, idx_buf, val_buf, rows, acc, sem):
    wid = lax.axis_index("s") * NC + lax.axis_index("c")
    b = wid >> 2
    tpair = (wid & 3) * 2
    iota = lax.iota(jnp.int32, L)

    pltpu.sync_copy(z_hbm.at[pl.ds(b * NF + tpair * D_SAE, CHUNK)], zbuf)

    pad_i = jnp.full((L,), tpair, jnp.int32)
    zero_f = jnp.zeros((L,), jnp.float32)
    for j in range(NGMAX + 1):
        idx_buf[j, :] = pad_i
        val_buf[pl.ds(j * L, L)] = zero_f

    U = 4

    def cbody(q, off):
        vs = [zbuf[pl.ds((q * U + i) * L, L)] for i in range(U)]
        ms = [v > 0.0 for v in vs]
        mo = ms[0]
        for i in range(1, U):
            mo = mo | ms[i]
        anyb = plsc.all_reduce_population_count(mo)[0] > 0

        def nonempty(o):
            for i in range(U):
                l_ids = (q * U + i) * L + iota
                r_ids = (l_ids & (D_SAE - 1)) * T + tpair + (l_ids >> 12)
                pos = o + plsc.cumsum(ms[i].astype(jnp.int32)) - 1
                plsc.store_scatter(idx_buf, [pos // G, pos % G], r_ids, mask=ms[i])
                plsc.store_scatter(val_buf, [pos], vs[i], mask=ms[i])
                o = o + plsc.all_reduce_population_count(ms[i])[0]
            return o

        return lax.cond(anyb, nonempty, lambda o: o, off)

    off0 = lax.fori_loop(0, NVC // 2 // U, cbody, jnp.int32(0))
    padp = off0 + iota
    plsc.store_scatter(idx_buf, [padp // G, padp % G], pad_i)
    plsc.store_scatter(val_buf, [padp], zero_f)
    off0r = (off0 + (G - 1)) & ~(G - 1)
    ng0 = off0r >> 4
    NB = 8

    def fire(g2, gbase):
        pltpu.async_copy(
            w_hbm.at[idx_buf.at[gbase + g2]], rows.at[g2], sem
        )
        return gbase

    nf0 = jnp.minimum(ng0, NB)
    lax.fori_loop(0, nf0, fire, jnp.int32(0))

    off = lax.fori_loop(NVC // 2 // U, NVC // U, cbody, off0r)

    for c in range(2 * D_IN // L):
        acc[pl.ds(c * L, L)] = zero_f

    ngrp = (off + (G - 1)) // G

    def round_body(r, carry):
        gbase = r * NB
        gn = jnp.minimum(ngrp - gbase, NB)
        start = jnp.maximum(nf0 - gbase, 0)
        lax.fori_loop(start, gn, fire, gbase)

        dn = lax.GatherDimensionNumbers(
            offset_dims=(), collapsed_slice_dims=(0,), start_index_map=(0,)
        )

        def drain_acc(g2, c2):
            pltpu.make_async_copy(
                w_hbm.at[idx_buf.at[0]], rows.at[g2], sem
            ).wait()
            gg = gbase + g2
            tbase = jnp.where(gg >= ng0, D_IN, 0)
            vgrp = val_buf[pl.ds(gg * G, G)]
            vjs = [
                lax.gather(
                    vgrp, jnp.full((L, 1), j, jnp.int32), dn, (1,),
                    mode=lax.GatherScatterMode.PROMISE_IN_BOUNDS,
                )
                for j in range(G)
            ]
            for c in range(D_IN // L):
                a = acc[pl.ds(tbase + c * L, L)]
                for j in range(G):
                    a = a + vjs[j] * rows[g2, j, pl.ds(c * L, L)]
                acc[pl.ds(tbase + c * L, L)] = a
            return c2

        lax.fori_loop(0, gn, drain_acc, jnp.int32(0))
        return carry

    lax.fori_loop(0, (ngrp + (NB - 1)) // NB, round_body, jnp.int32(0))

    pltpu.sync_copy(acc, o_hbm.at[pl.ds((b * T + tpair) * D_IN, 2 * D_IN)])


def _decode_sc(z_flat, W_dec):
    z1d = z_flat.reshape(B * NF)
    w2d = W_dec.reshape(D_SAE * T, D_IN)
    mesh = plsc.VectorSubcoreMesh(core_axis_name="c", subcore_axis_name="s")
    fn = pl.kernel(
        _sc_dec_body,
        out_type=jax.ShapeDtypeStruct((B * T * D_IN,), jnp.float32),
        mesh=mesh,
        compiler_params=pltpu.CompilerParams(needs_layout_passes=False),
        scratch_types=[
            pltpu.VMEM((CHUNK,), jnp.float32),
            pltpu.VMEM((NGMAX + 1, G), jnp.int32),
            pltpu.VMEM((KMAX + G,), jnp.float32),
            pltpu.VMEM((8, G, D_IN), jnp.float32),
            pltpu.VMEM((2 * D_IN,), jnp.float32),
            pltpu.SemaphoreType.DMA,
        ],
    )
    return fn(z1d, w2d).reshape(B, T, D_IN)


def _fin_body(p_ref, b_ref, x_ref, xh_ref, loss_ref):
    xh = p_ref[...] + b_ref[...][None]
    xh_ref[...] = xh
    d = xh - x_ref[...]
    loss_ref[0, 0] = jnp.sum(d * d) / (B * T)


def _finalize(partial, b_dec, x):
    return pl.pallas_call(
        _fin_body,
        out_specs=(
            pl.BlockSpec(memory_space=pltpu.VMEM),
            pl.BlockSpec(memory_space=pltpu.SMEM),
        ),
        out_shape=(
            jax.ShapeDtypeStruct((B, T, D_IN), jnp.float32),
            jax.ShapeDtypeStruct((1, 1), jnp.float32),
        ),
    )(partial, b_dec, x)


def kernel(x, W_enc, W_dec, b_enc, b_dec, k):
    b_enc2 = b_enc.reshape(1, D_SAE)
    k_arr = jnp.asarray(k, jnp.int32).reshape(1)
    z = _encode_topk(x, W_enc, b_enc2, k_arr)
    partial = _decode_sc(z.reshape(B, NF), W_dec)
    x_hat, loss = _finalize(partial, b_dec, x)
    return (loss.reshape(()), x_hat, z)
